# baseline (speedup 1.0000x reference)
import functools

import jax
import jax.numpy as jnp
from jax import lax
from jax.experimental import pallas as pl
from jax.experimental.pallas import tpu as pltpu

_C_BLK = 2048


def _stream_body(t_ref, x_ref, loss_ref, m_ref, s_ref, p_ref, *, c_blk,
                 v_total, n_blk):
    j = pl.program_id(0)

    @pl.when(j == 0)
    def _():
        m_ref[...] = jnp.full(m_ref.shape, -jnp.inf, m_ref.dtype)
        s_ref[...] = jnp.zeros(s_ref.shape, s_ref.dtype)
        p_ref[...] = jnp.zeros(p_ref.shape, p_ref.dtype)

    x = x_ref[...]
    col = lax.broadcasted_iota(jnp.int32, x.shape, 1) + j * c_blk
    xm = jnp.where(col < v_total, x, -jnp.inf)
    m_old = m_ref[...]
    m_new = jnp.maximum(m_old, jnp.max(xm, axis=1, keepdims=True))
    e = jnp.exp(xm - m_new)
    s_ref[...] = s_ref[...] * jnp.exp(m_old - m_new) + jnp.sum(
        e, axis=1, keepdims=True)
    m_ref[...] = m_new
    p_ref[...] += jnp.sum(jnp.where(col == t_ref[...], x, 0.0), axis=1,
                          keepdims=True)

    @pl.when(j == n_blk - 1)
    def _():
        loss_ref[...] = m_ref[...] + jnp.log(s_ref[...]) - p_ref[...]


def _topk_body(loss_ref, out_ref, *, k):
    loss = loss_ref[...]
    b = lax.bitcast_convert_type(loss, jnp.int32)
    key = b ^ (lax.shift_right_arithmetic(b, 31) & jnp.int32(0x7FFFFFFF))

    def cnt_ge(thresh):
        return jnp.sum((key >= thresh).astype(jnp.int32))

    base0 = jnp.where(cnt_ge(jnp.int32(0)) >= k, jnp.int32(0),
                      jnp.int32(-(2**31)))

    def body(i, base):
        cand = base | lax.shift_left(jnp.int32(1), 30 - i)
        return jnp.where(cnt_ge(cand) >= k, cand, base)

    big_t = lax.fori_loop(0, 31, body, base0)
    tb = big_t ^ (lax.shift_right_arithmetic(big_t, 31) & jnp.int32(0x7FFFFFFF))
    tval = lax.bitcast_convert_type(tb, jnp.float32)
    gt = loss > tval
    cnt_gt = jnp.sum(gt.astype(jnp.float32))
    sum_gt = jnp.sum(jnp.where(gt, loss, 0.0))
    res = (sum_gt + (jnp.float32(k) - cnt_gt) * tval) / jnp.float32(k)
    out_ref[...] = jnp.full((1, 1), res, jnp.float32)


@jax.jit
def kernel(inputs, targets):
    n, v = inputs.shape
    k = int(0.25 * n)
    c_blk = _C_BLK
    n_blk = (v + c_blk - 1) // c_blk
    t2 = targets.reshape(n, 1).astype(jnp.int32)
    loss = pl.pallas_call(
        functools.partial(_stream_body, c_blk=c_blk, v_total=v, n_blk=n_blk),
        grid=(n_blk,),
        in_specs=[
            pl.BlockSpec((n, 1), lambda j: (0, 0)),
            pl.BlockSpec((n, c_blk), lambda j: (0, j)),
        ],
        out_specs=pl.BlockSpec((n, 1), lambda j: (0, 0)),
        out_shape=jax.ShapeDtypeStruct((n, 1), jnp.float32),
        scratch_shapes=[
            pltpu.VMEM((n, 1), jnp.float32),
            pltpu.VMEM((n, 1), jnp.float32),
            pltpu.VMEM((n, 1), jnp.float32),
        ],
        compiler_params=pltpu.CompilerParams(
            dimension_semantics=("arbitrary",)),
    )(t2, inputs)
    loss8 = loss.reshape(8, n // 8)
    out = pl.pallas_call(
        functools.partial(_topk_body, k=k),
        out_shape=jax.ShapeDtypeStruct((1, 1), jnp.float32),
    )(loss8)
    return out[0, 0]
